# baseline (speedup 1.0000x reference)
import functools

import numpy as np
import jax
import jax.numpy as jnp
from jax import lax
from jax.experimental import pallas as pl
from jax.experimental.pallas import tpu as pltpu
from jax.experimental.pallas import tpu_sc as plsc

_BINS = (512, 128, 256, 256, 64, 256, 256, 16, 256, 64, 16, 128, 64, 128, 64, 512, 512)
_EMB = 16
_NF = 17
_BATCH = 4096
_EPS = 1e-5
_OFFS = np.concatenate([[0], np.cumsum(_BINS)[:-1]]).astype(np.int32)
_VOCAB = int(np.sum(_BINS))

_NC, _NS = 2, 16
_NW = _NC * _NS
_ROWS_PW = _BATCH // _NW
_WIDTH = _NF * _EMB
_LTILES = (_WIDTH + 127) // 128
_SLAB = _ROWS_PW * _LTILES * 8 * 128 // 8
_OUT_FLAT = _NW * _SLAB


_IPW = _ROWS_PW * 24
_NCHUNK = _IPW // 128
_XPW = _ROWS_PW * _NF

_DST0 = np.array([(i // 8) * 64 + i % 8 for i in range(16)], np.int32)


def _gather_body(table_hbm, x_hbm, out_hbm, x_v, idx_v, rows_v, sem, sem2):
    wid = lax.axis_index("s") * _NC + lax.axis_index("c")
    pltpu.sync_copy(x_hbm.at[pl.ds(wid * _XPW, _XPW)], x_v)

    iota = lax.iota(jnp.int32, 16)
    dst0 = (iota // 8) * 64 + lax.rem(iota, 8)
    lane15 = iota == 15
    zeros = jnp.zeros((16,), jnp.int32)

    def zrow(k, _):
        for t in range(8):
            idx_v[k, pl.ds(t * 16, 16)] = iota + t * 16
        return _

    lax.fori_loop(0, _NCHUNK, zrow, None)

    def prow(r, _):
        base = (r // 8) * 192 + lax.rem(r, 8) * 8
        m0 = dst0 + base
        f015 = x_v[pl.ds(r * _NF, 16)]
        plsc.store_scatter(idx_v, [m0 >> 7, m0 & 127], f015)
        m1 = jnp.full((16,), base + 128, jnp.int32)
        f16 = x_v[pl.ds(r * _NF + 1, 16)]
        plsc.store_scatter(idx_v, [m1 >> 7, m1 & 127], f16, mask=lane15)
        return _

    lax.fori_loop(0, _ROWS_PW, prow, None)

    copies = [
        pltpu.async_copy(table_hbm.at[idx_v.at[k]], rows_v.at[k], sem)
        for k in range(_NCHUNK)
    ]
    wbs = []
    for k, c in enumerate(copies):
        c.wait()
        wbs.append(pltpu.async_copy(
            rows_v.at[k], out_hbm.at[pl.ds(wid * _IPW + k * 128, 128), :],
            sem2))
    for w in wbs:
        w.wait()


@functools.lru_cache(maxsize=None)
def _sc_gather():
    return pl.kernel(
        _gather_body,
        out_type=jax.ShapeDtypeStruct((_NW * _IPW, _EMB), jnp.float32),
        mesh=plsc.VectorSubcoreMesh(core_axis_name="c", subcore_axis_name="s",
                                    num_cores=_NC, num_subcores=_NS),
        scratch_types=[
            pltpu.VMEM((_XPW,), jnp.int32),
            pltpu.VMEM((_NCHUNK, 128), jnp.int32),
            pltpu.VMEM((_NCHUNK, 128, _EMB), jnp.float32),
            pltpu.SemaphoreType.DMA,
            pltpu.SemaphoreType.DMA,
        ],
        compiler_params=pltpu.CompilerParams(use_tc_tiling_on_sc=False,
                                             needs_layout_passes=False),
    )


def _pack_body(*refs):
    out_ref = refs[-1]
    for i in range(_NF):
        out_ref[pl.ds(int(_OFFS[i]), _BINS[i]), :] = refs[i][:]


_pack = pl.pallas_call(
    _pack_body,
    out_shape=jax.ShapeDtypeStruct((_VOCAB, _EMB), jnp.float32),
)


def _mlp_body(x4_ref, w0, b0, g0, be0, w1, b1, g1, be1, w2, b2, g2, be2,
              w3, b3, out_ref):
    x4 = x4_ref[:]
    parts = [x4[:, c, :, :].reshape(_BATCH, 128) for c in range(_LTILES)]
    h = jnp.concatenate(parts, axis=1)[:, :_WIDTH]

    def layer(h, w, b, g, be):
        h = jnp.dot(h, w[:], preferred_element_type=jnp.float32) + b[:]
        h = jnp.maximum(h, 0.0)
        m = jnp.mean(h, axis=0, keepdims=True)
        v = jnp.mean((h - m) ** 2, axis=0, keepdims=True)
        return (h - m) * (g[:] * lax.rsqrt(v + _EPS)) + be[:]

    h = layer(h, w0, b0, g0, be0)
    h = layer(h, w1, b1, g1, be1)
    h = layer(h, w2, b2, g2, be2)
    out_ref[:] = jnp.dot(h, w3[:], preferred_element_type=jnp.float32) + b3[:]


_mlp = pl.pallas_call(
    _mlp_body,
    out_shape=jax.ShapeDtypeStruct((_BATCH, 1), jnp.float32),
)


def kernel(x, emb_0, emb_1, emb_2, emb_3, emb_4, emb_5, emb_6, emb_7, emb_8,
           emb_9, emb_10, emb_11, emb_12, emb_13, emb_14, emb_15, emb_16,
           W0, b0, W1, b1, W2, b2, W3, b3, g0, beta0, g1, beta1, g2, beta2):
    embs = [emb_0, emb_1, emb_2, emb_3, emb_4, emb_5, emb_6, emb_7, emb_8,
            emb_9, emb_10, emb_11, emb_12, emb_13, emb_14, emb_15, emb_16]
    table = _pack(*embs)
    xo = (x + _OFFS[None, :]).reshape(-1)
    rows = _sc_gather()(table, xo)
    x4 = rows.reshape(_BATCH // 8, _LTILES, 8, 128)
    r = lambda a: a.reshape(1, -1)
    out = _mlp(x4, W0, r(b0), r(g0), r(beta0), W1, r(b1), r(g1), r(beta1),
               W2, r(b2), r(g2), r(beta2), W3, r(b3))
    return out


# speedup vs baseline: 1.3074x; 1.3074x over previous
import functools

import numpy as np
import jax
import jax.numpy as jnp
from jax import lax
from jax.experimental import pallas as pl
from jax.experimental.pallas import tpu as pltpu
from jax.experimental.pallas import tpu_sc as plsc

_BINS = (512, 128, 256, 256, 64, 256, 256, 16, 256, 64, 16, 128, 64, 128, 64, 512, 512)
_EMB = 16
_NF = 17
_BATCH = 4096
_EPS = 1e-5
_OFFS = np.concatenate([[0], np.cumsum(_BINS)[:-1]]).astype(np.int32)
_VOCAB = int(np.sum(_BINS))

_NC, _NS = 2, 16
_NW = _NC * _NS
_ROWS_PW = _BATCH // _NW
_WIDTH = _NF * _EMB
_LTILES = (_WIDTH + 127) // 128
_IPW = _ROWS_PW * 24
_NCHUNK = _IPW // 128
_XPW = _ROWS_PW * _NF


def _gather_body(table_hbm, x_hbm, out_hbm, x_v, idx_v, rows_v, sem, sem2):
    wid = lax.axis_index("s") * _NC + lax.axis_index("c")
    pltpu.sync_copy(x_hbm.at[pl.ds(wid * _XPW, _XPW)], x_v)

    iota = lax.iota(jnp.int32, 16)
    dst0 = (iota // 8) * 64 + lax.rem(iota, 8)
    lane15 = iota == 15

    def zrow(k, _):
        for t in range(8):
            idx_v[k, pl.ds(t * 16, 16)] = iota + t * 16
        return _

    lax.fori_loop(0, _NCHUNK, zrow, None)

    def prow(r, _):
        base = (r // 8) * 192 + lax.rem(r, 8) * 8
        m0 = dst0 + base
        f015 = x_v[pl.ds(r * _NF, 16)]
        plsc.store_scatter(idx_v, [m0 >> 7, m0 & 127], f015)
        m1 = jnp.full((16,), base + 128, jnp.int32)
        f16 = x_v[pl.ds(r * _NF + 1, 16)]
        plsc.store_scatter(idx_v, [m1 >> 7, m1 & 127], f16, mask=lane15)
        return _

    lax.fori_loop(0, _ROWS_PW, prow, None)

    copies = [
        pltpu.async_copy(table_hbm.at[idx_v.at[k]], rows_v.at[k], sem)
        for k in range(_NCHUNK)
    ]
    wbs = []
    for k, c in enumerate(copies):
        c.wait()
        wbs.append(pltpu.async_copy(
            rows_v.at[k], out_hbm.at[pl.ds(wid * _IPW + k * 128, 128), :],
            sem2))
    for w in wbs:
        w.wait()


@functools.lru_cache(maxsize=None)
def _sc_gather():
    return pl.kernel(
        _gather_body,
        out_type=jax.ShapeDtypeStruct((_NW * _IPW, _EMB), jnp.float32),
        mesh=plsc.VectorSubcoreMesh(core_axis_name="c", subcore_axis_name="s",
                                    num_cores=_NC, num_subcores=_NS),
        scratch_types=[
            pltpu.VMEM((_XPW,), jnp.int32),
            pltpu.VMEM((_NCHUNK, 128), jnp.int32),
            pltpu.VMEM((_NCHUNK, 128, _EMB), jnp.float32),
            pltpu.SemaphoreType.DMA,
            pltpu.SemaphoreType.DMA,
        ],
        compiler_params=pltpu.CompilerParams(use_tc_tiling_on_sc=False,
                                             needs_layout_passes=False),
    )


def _mlp_body(x4_ref, w0, b0, g0, be0, w1, b1, g1, be1, w2, b2, g2, be2,
              w3, b3, out_ref):
    x4 = x4_ref[:]
    parts = [x4[:, c, :, :].reshape(_BATCH, 128) for c in range(_LTILES)]
    h = jnp.concatenate(parts, axis=1)[:, :_WIDTH]

    def layer(h, w, b, g, be):
        h = jnp.dot(h, w[:], preferred_element_type=jnp.float32) + b[:]
        h = jnp.maximum(h, 0.0)
        m = jnp.mean(h, axis=0, keepdims=True)
        v = jnp.mean((h - m) ** 2, axis=0, keepdims=True)
        return (h - m) * (g[:] * lax.rsqrt(v + _EPS)) + be[:]

    h = layer(h, w0, b0, g0, be0)
    h = layer(h, w1, b1, g1, be1)
    h = layer(h, w2, b2, g2, be2)
    out_ref[:] = jnp.dot(h, w3[:], preferred_element_type=jnp.float32) + b3[:]


_mlp = pl.pallas_call(
    _mlp_body,
    out_shape=jax.ShapeDtypeStruct((_BATCH, 1), jnp.float32),
)


def kernel(x, emb_0, emb_1, emb_2, emb_3, emb_4, emb_5, emb_6, emb_7, emb_8,
           emb_9, emb_10, emb_11, emb_12, emb_13, emb_14, emb_15, emb_16,
           W0, b0, W1, b1, W2, b2, W3, b3, g0, beta0, g1, beta1, g2, beta2):
    embs = [emb_0, emb_1, emb_2, emb_3, emb_4, emb_5, emb_6, emb_7, emb_8,
            emb_9, emb_10, emb_11, emb_12, emb_13, emb_14, emb_15, emb_16]
    table = jnp.concatenate(embs, axis=0)
    xo = (x + _OFFS[None, :]).reshape(-1)
    rows = _sc_gather()(table, xo)
    x4 = rows.reshape(_BATCH // 8, _LTILES, 8, 128)
    r = lambda a: a.reshape(1, -1)
    out = _mlp(x4, W0, r(b0), r(g0), r(beta0), W1, r(b1), r(g1), r(beta1),
               W2, r(b2), r(g2), r(beta2), W3, r(b3))
    return out
